# baseline (speedup 1.0000x reference)
import jax, jax.numpy as jnp
import numpy as np
from jax.experimental import pallas as pl

N = 100000
D = 128
DL = 16
S = 16
B = 8192
EMB = 128
H = 4
L = 2
FF = 256
C = 40


def _layernorm(z):
    m = jnp.mean(z, axis=-1, keepdims=True)
    v = jnp.var(z, axis=-1, keepdims=True)
    return (z - m) / jnp.sqrt(v + 1e-5)


def kernel(nodes, neigh, feat, lap, W_in, Wq, Wk, Wv, Wo, W1, W2, W_dense, b_dense):
    nb = jnp.take(neigh, nodes, axis=0)
    tok = jnp.concatenate([nodes[:, None], nb], axis=1)
    T = S + 1
    dh = EMB // H
    fx = jnp.take(feat, tok, axis=0)
    lx = jnp.take(lap, tok, axis=0)
    x = jnp.concatenate([fx, lx], axis=-1) @ W_in
    Bsz = x.shape[0]
    for l in range(L):
        q = (x @ Wq[l]).reshape(Bsz, T, H, dh).transpose(0, 2, 1, 3)
        k = (x @ Wk[l]).reshape(Bsz, T, H, dh).transpose(0, 2, 1, 3)
        v = (x @ Wv[l]).reshape(Bsz, T, H, dh).transpose(0, 2, 1, 3)
        att = jax.nn.softmax(jnp.einsum('bhqd,bhkd->bhqk', q, k) / np.sqrt(dh), axis=-1)
        o = jnp.einsum('bhqk,bhkd->bhqd', att, v).transpose(0, 2, 1, 3).reshape(Bsz, T, EMB)
        x = _layernorm(x + o @ Wo[l])
        f = jax.nn.relu(x @ W1[l]) @ W2[l]
        x = _layernorm(x + f)
    h = x[:, 0, :]
    scores = h @ W_dense + b_dense
    return scores


# speedup vs baseline: 5.1716x; 5.1716x over previous
import functools

import jax
import jax.numpy as jnp
import numpy as np
from jax import lax
from jax.experimental import pallas as pl
from jax.experimental.pallas import tpu as pltpu
from jax.experimental.pallas import tpu_sc as plsc

D = 128
DL = 16
S = 16
T = S + 1
EMB = 128
H = 4
DH = EMB // H
NLAYER = 2
FF = 256
C = 40


def _sc_mesh():
    return plsc.VectorSubcoreMesh(core_axis_name="c", subcore_axis_name="s")


def _worker_info(B):
    info = plsc.get_sparse_core_info()
    NW = info.num_cores * info.num_subcores
    return info.num_cores, NW, B // NW


def _sc_gather_nb(nodes, neigh_wide):
    B = nodes.shape[0]
    NC, NW, BW = _worker_info(B)
    NCHUNK = BW // 128

    @functools.partial(
        pl.kernel,
        mesh=_sc_mesh(),
        out_type=jax.ShapeDtypeStruct((B, 128), jnp.int32),
        scratch_types=[
            pltpu.VMEM((BW,), jnp.int32),
            pltpu.VMEM((BW,), jnp.int32),
            pltpu.VMEM((128, 128), jnp.int32),
            pltpu.SemaphoreType.DMA,
        ],
    )
    def nb_kernel(nodes_hbm, neigh_hbm, nb_hbm, seeds_v, idx_v, buf, sem):
        wid = lax.axis_index("s") * NC + lax.axis_index("c")
        base = wid * BW
        pltpu.sync_copy(nodes_hbm.at[pl.ds(base, BW)], seeds_v)
        for i in range(BW // 16):
            sl = pl.ds(i * 16, 16)
            idx_v[sl] = lax.shift_right_logical(seeds_v[sl], 3)
        for c in range(NCHUNK):
            pltpu.async_copy(
                neigh_hbm.at[idx_v.at[pl.ds(c * 128, 128)]], buf, sem).wait()
            pltpu.sync_copy(buf, nb_hbm.at[pl.ds(base + c * 128, 128), :])

    return nb_kernel(nodes, neigh_wide)


def _tok_body(nodes_ref, nbw_ref, out_ref):
    nodes = nodes_ref[...]
    offs = jnp.transpose(nodes, (1, 0)) & 7
    nb = nbw_ref[:, 0:S]
    for k in range(1, 8):
        nb = jnp.where(offs == k, nbw_ref[:, k * S:(k + 1) * S], nb)
    nT = jnp.transpose(nb, (1, 0))
    out_ref[:, 0, :] = jnp.concatenate([nodes, nT], axis=0)


def _tc_tokens(nodes2d, nbwide):
    B = nbwide.shape[0]
    BB = 256
    return pl.pallas_call(
        _tok_body,
        grid=(B // BB,),
        in_specs=[
            pl.BlockSpec((1, BB), lambda i: (0, i)),
            pl.BlockSpec((BB, 128), lambda i: (i, 0)),
        ],
        out_specs=pl.BlockSpec((T, 1, BB), lambda i: (0, 0, i)),
        out_shape=jax.ShapeDtypeStruct((T, 1, B), jnp.int32),
    )(nodes2d, nbwide)


def _sc_gather_rows(tokT, feat, lap_pad):
    B = tokT.shape[2]
    NC, NW, BW = _worker_info(B)
    NCHUNK = BW // 128

    @functools.partial(
        pl.kernel,
        mesh=_sc_mesh(),
        out_type=[
            jax.ShapeDtypeStruct((T, B, D), jnp.float32),
            jax.ShapeDtypeStruct((T, B, D), jnp.float32),
        ],
        scratch_types=[
            pltpu.VMEM((T * BW,), jnp.int32),
            pltpu.VMEM((2, 128, D), jnp.float32),
            pltpu.VMEM((2, 128, D), jnp.float32),
            pltpu.SemaphoreType.DMA,
            pltpu.SemaphoreType.DMA,
        ],
    )
    def row_kernel(tokT_hbm, feat_hbm, lap_hbm, gf_hbm, gl_hbm,
                   tok_v, fbuf, lbuf, fsem, lsem):
        wid = lax.axis_index("s") * NC + lax.axis_index("c")
        base = wid * BW
        for t in range(T):
            pltpu.sync_copy(tokT_hbm.at[t, 0, pl.ds(base, BW)],
                            tok_v.at[pl.ds(t * BW, BW)])
        for t in range(T):
            for c in range(NCHUNK):
                idx = tok_v.at[pl.ds(t * BW + c * 128, 128)]
                sl = c & 1
                pltpu.async_copy(feat_hbm.at[idx], fbuf.at[sl], fsem).wait()
                pltpu.sync_copy(fbuf.at[sl],
                                gf_hbm.at[t, pl.ds(base + c * 128, 128), :])
                pltpu.async_copy(lap_hbm.at[idx], lbuf.at[sl], lsem).wait()
                pltpu.sync_copy(lbuf.at[sl],
                                gl_hbm.at[t, pl.ds(base + c * 128, 128), :])

    return row_kernel(tokT, feat, lap_pad)


def _layernorm_t(z):
    m = jnp.mean(z, axis=0, keepdims=True)
    zc = z - m
    v = jnp.mean(zc * zc, axis=0, keepdims=True)
    return zc * lax.rsqrt(v + 1e-5)


def _tc_body(gf_ref, gl_ref, winf_ref, winl_ref, wq_ref, wk_ref, wv_ref,
             wo_ref, w1_ref, w2_ref, wd_ref, bd_ref, out_ref):
    BB = gf_ref.shape[1]
    R = T * BB
    scale = np.float32(1.0 / np.sqrt(DH))
    f32 = jnp.float32

    gf = gf_ref[...].reshape(R, D)
    gl = gl_ref[...].reshape(R, D)
    x = (jnp.dot(gf, winf_ref[...], preferred_element_type=f32)
         + jnp.dot(gl, winl_ref[...], preferred_element_type=f32))
    xT = jnp.transpose(x, (1, 0))

    for l in range(NLAYER):
        qT = jnp.dot(wq_ref[l], xT, preferred_element_type=f32)
        kT = jnp.dot(wk_ref[l], xT, preferred_element_type=f32)
        vT = jnp.dot(wv_ref[l], xT, preferred_element_type=f32)
        head_cols = []
        for h in range(H):
            qh = qT[h * DH:(h + 1) * DH, :]
            kh = kT[h * DH:(h + 1) * DH, :]
            vh = vT[h * DH:(h + 1) * DH, :]
            vcols = [vh[:, j * BB:(j + 1) * BB] for j in range(T)]
            icols = []
            for i in range(T):
                qhi = qh[:, i * BB:(i + 1) * BB]
                rows = [jnp.sum(qhi * kh[:, j * BB:(j + 1) * BB],
                                axis=0, keepdims=True) for j in range(T)]
                s = jnp.concatenate(rows, axis=0) * scale
                m = jnp.max(s, axis=0, keepdims=True)
                e = jnp.exp(s - m)
                w = e / jnp.sum(e, axis=0, keepdims=True)
                acc = w[0:1, :] * vcols[0]
                for j in range(1, T):
                    acc = acc + w[j:j + 1, :] * vcols[j]
                icols.append(acc)
            head_cols.append(icols)
        oT = jnp.concatenate(
            [jnp.concatenate(head_cols[h], axis=1) for h in range(H)], axis=0)
        xT = _layernorm_t(xT + jnp.dot(wo_ref[l], oT,
                                       preferred_element_type=f32))
        f = jnp.dot(w2_ref[l],
                    jax.nn.relu(jnp.dot(w1_ref[l], xT,
                                        preferred_element_type=f32)),
                    preferred_element_type=f32)
        xT = _layernorm_t(xT + f)

    h0 = jnp.transpose(xT[:, :BB], (1, 0))
    out_ref[...] = (jnp.dot(h0, wd_ref[...], preferred_element_type=f32)
                    + bd_ref[...])


def _tc_forward(gf, gl, W_inf, W_inl, Wq, Wk, Wv, Wo, W1, W2, W_dense,
                b_dense2d, interpret=False):
    B = gf.shape[1]
    BB = 256
    grid = B // BB
    full = lambda shp: pl.BlockSpec(shp, lambda i: tuple(0 for _ in shp))
    return pl.pallas_call(
        _tc_body,
        grid=(grid,),
        in_specs=[
            pl.BlockSpec((T, BB, D), lambda i: (0, i, 0)),
            pl.BlockSpec((T, BB, D), lambda i: (0, i, 0)),
            full((D, EMB)),
            full((D, EMB)),
            full((NLAYER, EMB, EMB)),
            full((NLAYER, EMB, EMB)),
            full((NLAYER, EMB, EMB)),
            full((NLAYER, EMB, EMB)),
            full((NLAYER, FF, EMB)),
            full((NLAYER, EMB, FF)),
            full((EMB, C)),
            full((1, C)),
        ],
        out_specs=pl.BlockSpec((BB, C), lambda i: (i, 0)),
        out_shape=jax.ShapeDtypeStruct((B, C), jnp.float32),
        interpret=interpret,
    )(gf, gl, W_inf, W_inl, Wq, Wk, Wv, Wo, W1, W2, W_dense, b_dense2d)


def kernel(nodes, neigh, feat, lap, W_in, Wq, Wk, Wv, Wo, W1, W2,
           W_dense, b_dense):
    nodes = nodes.astype(jnp.int32)
    neigh = neigh.astype(jnp.int32)
    B = nodes.shape[0]
    neigh_wide = neigh.reshape(neigh.shape[0] * S // 128, 128)
    lap_pad = jnp.pad(lap, ((0, 0), (0, D - DL)))
    W_inf = W_in[:D]
    W_inl = jnp.pad(W_in[D:], ((0, D - DL), (0, 0)))
    nbwide = _sc_gather_nb(nodes, neigh_wide)
    tokT = _tc_tokens(nodes.reshape(1, B), nbwide)
    gf, gl = _sc_gather_rows(tokT, feat, lap_pad)
    return _tc_forward(gf, gl, W_inf, W_inl,
                       jnp.swapaxes(Wq, 1, 2), jnp.swapaxes(Wk, 1, 2),
                       jnp.swapaxes(Wv, 1, 2), jnp.swapaxes(Wo, 1, 2),
                       jnp.swapaxes(W1, 1, 2), jnp.swapaxes(W2, 1, 2),
                       W_dense, b_dense.reshape(1, C))


# speedup vs baseline: 5.5617x; 1.0754x over previous
import functools

import jax
import jax.numpy as jnp
import numpy as np
from jax import lax
from jax.experimental import pallas as pl
from jax.experimental.pallas import tpu as pltpu
from jax.experimental.pallas import tpu_sc as plsc

D = 128
DL = 16
S = 16
T = S + 1
EMB = 128
H = 4
DH = EMB // H
NLAYER = 2
FF = 256
C = 40


def _sc_mesh():
    return plsc.VectorSubcoreMesh(core_axis_name="c", subcore_axis_name="s")


def _worker_info(B):
    info = plsc.get_sparse_core_info()
    NW = info.num_cores * info.num_subcores
    return info.num_cores, NW, B // NW


def _sc_gather_nb(nodes, neigh_wide):
    B = nodes.shape[0]
    NC, NW, BW = _worker_info(B)
    NCHUNK = BW // 128

    @functools.partial(
        pl.kernel,
        mesh=_sc_mesh(),
        out_type=jax.ShapeDtypeStruct((B, 128), jnp.int32),
        scratch_types=[
            pltpu.VMEM((BW,), jnp.int32),
            pltpu.VMEM((BW,), jnp.int32),
            pltpu.VMEM((128, 128), jnp.int32),
            pltpu.SemaphoreType.DMA,
        ],
    )
    def nb_kernel(nodes_hbm, neigh_hbm, nb_hbm, seeds_v, idx_v, buf, sem):
        wid = lax.axis_index("s") * NC + lax.axis_index("c")
        base = wid * BW
        pltpu.sync_copy(nodes_hbm.at[pl.ds(base, BW)], seeds_v)
        for i in range(BW // 16):
            sl = pl.ds(i * 16, 16)
            idx_v[sl] = lax.shift_right_logical(seeds_v[sl], 3)
        for c in range(NCHUNK):
            pltpu.async_copy(
                neigh_hbm.at[idx_v.at[pl.ds(c * 128, 128)]], buf, sem).wait()
            pltpu.sync_copy(buf, nb_hbm.at[pl.ds(base + c * 128, 128), :])

    return nb_kernel(nodes, neigh_wide)


def _tok_body(nodes_ref, nbw_ref, out_ref):
    nodes = nodes_ref[...]
    offs = jnp.transpose(nodes, (1, 0)) & 7
    nb = nbw_ref[:, 0:S]
    for k in range(1, 8):
        nb = jnp.where(offs == k, nbw_ref[:, k * S:(k + 1) * S], nb)
    nT = jnp.transpose(nb, (1, 0))
    out_ref[:, 0, :] = jnp.concatenate([nodes, nT], axis=0)


def _tc_tokens(nodes2d, nbwide):
    B = nbwide.shape[0]
    BB = 256
    return pl.pallas_call(
        _tok_body,
        grid=(B // BB,),
        in_specs=[
            pl.BlockSpec((1, BB), lambda i: (0, i)),
            pl.BlockSpec((BB, 128), lambda i: (i, 0)),
        ],
        out_specs=pl.BlockSpec((T, 1, BB), lambda i: (0, 0, i)),
        out_shape=jax.ShapeDtypeStruct((T, 1, B), jnp.int32),
    )(nodes2d, nbwide)


def _sc_gather_rows(tokT, feat, lap_pad):
    B = tokT.shape[2]
    NC, NW, BW = _worker_info(B)
    NCHUNK = BW // 128

    @functools.partial(
        pl.kernel,
        mesh=_sc_mesh(),
        out_type=[
            jax.ShapeDtypeStruct((T, B, D), jnp.float32),
            jax.ShapeDtypeStruct((T, B, D), jnp.float32),
        ],
        scratch_types=[
            pltpu.VMEM((T * BW,), jnp.int32),
            pltpu.VMEM((2, 128, D), jnp.float32),
            pltpu.VMEM((2, 128, D), jnp.float32),
            pltpu.SemaphoreType.DMA,
            pltpu.SemaphoreType.DMA,
            pltpu.SemaphoreType.DMA,
            pltpu.SemaphoreType.DMA,
        ],
    )
    def row_kernel(tokT_hbm, feat_hbm, lap_hbm, gf_hbm, gl_hbm,
                   tok_v, fbuf, lbuf, fsem, lsem, fwsem, lwsem):
        wid = lax.axis_index("s") * NC + lax.axis_index("c")
        base = wid * BW
        for t in range(T):
            pltpu.sync_copy(tokT_hbm.at[t, 0, pl.ds(base, BW)],
                            tok_v.at[pl.ds(t * BW, BW)])
        steps = [(t, c) for t in range(T) for c in range(NCHUNK)]
        n = len(steps)
        fg = [None] * n
        lg = [None] * n
        fw = [None] * n
        lw = [None] * n

        def start_wb(s):
            t, c = steps[s]
            b = s & 1
            dst = pl.ds(base + c * 128, 128)
            fg[s].wait()
            fw[s] = pltpu.async_copy(fbuf.at[b], gf_hbm.at[t, dst, :], fwsem)
            lg[s].wait()
            lw[s] = pltpu.async_copy(lbuf.at[b], gl_hbm.at[t, dst, :], lwsem)

        for s, (t, c) in enumerate(steps):
            b = s & 1
            if s >= 2:
                fw[s - 2].wait()
                lw[s - 2].wait()
            idx = tok_v.at[pl.ds(t * BW + c * 128, 128)]
            fg[s] = pltpu.async_copy(feat_hbm.at[idx], fbuf.at[b], fsem)
            lg[s] = pltpu.async_copy(lap_hbm.at[idx], lbuf.at[b], lsem)
            if s >= 1:
                start_wb(s - 1)
        start_wb(n - 1)
        fw[n - 2].wait()
        lw[n - 2].wait()
        fw[n - 1].wait()
        lw[n - 1].wait()

    return row_kernel(tokT, feat, lap_pad)


def _layernorm_t(z):
    m = jnp.mean(z, axis=0, keepdims=True)
    zc = z - m
    v = jnp.mean(zc * zc, axis=0, keepdims=True)
    return zc * lax.rsqrt(v + 1e-5)


def _tc_body(gf_ref, gl_ref, winf_ref, winl_ref, wq_ref, wk_ref, wv_ref,
             wo_ref, w1_ref, w2_ref, wd_ref, bd_ref, out_ref):
    BB = gf_ref.shape[1]
    R = T * BB
    scale = np.float32(1.0 / np.sqrt(DH))
    f32 = jnp.float32

    gf = gf_ref[...].reshape(R, D)
    gl = gl_ref[...].reshape(R, D)
    x = (jnp.dot(gf, winf_ref[...], preferred_element_type=f32)
         + jnp.dot(gl, winl_ref[...], preferred_element_type=f32))
    xT = jnp.transpose(x, (1, 0))

    for l in range(NLAYER):
        qT = jnp.dot(wq_ref[l], xT, preferred_element_type=f32)
        kT = jnp.dot(wk_ref[l], xT, preferred_element_type=f32)
        vT = jnp.dot(wv_ref[l], xT, preferred_element_type=f32)
        head_cols = []
        for h in range(H):
            qh = qT[h * DH:(h + 1) * DH, :]
            kh = kT[h * DH:(h + 1) * DH, :]
            vh = vT[h * DH:(h + 1) * DH, :]
            vcols = [vh[:, j * BB:(j + 1) * BB] for j in range(T)]
            icols = []
            for i in range(T):
                qhi = qh[:, i * BB:(i + 1) * BB]
                rows = [jnp.sum(qhi * kh[:, j * BB:(j + 1) * BB],
                                axis=0, keepdims=True) for j in range(T)]
                s = jnp.concatenate(rows, axis=0) * scale
                m = jnp.max(s, axis=0, keepdims=True)
                e = jnp.exp(s - m)
                w = e / jnp.sum(e, axis=0, keepdims=True)
                acc = w[0:1, :] * vcols[0]
                for j in range(1, T):
                    acc = acc + w[j:j + 1, :] * vcols[j]
                icols.append(acc)
            head_cols.append(icols)
        oT = jnp.concatenate(
            [jnp.concatenate(head_cols[h], axis=1) for h in range(H)], axis=0)
        xT = _layernorm_t(xT + jnp.dot(wo_ref[l], oT,
                                       preferred_element_type=f32))
        f = jnp.dot(w2_ref[l],
                    jax.nn.relu(jnp.dot(w1_ref[l], xT,
                                        preferred_element_type=f32)),
                    preferred_element_type=f32)
        xT = _layernorm_t(xT + f)

    h0 = jnp.transpose(xT[:, :BB], (1, 0))
    out_ref[...] = (jnp.dot(h0, wd_ref[...], preferred_element_type=f32)
                    + bd_ref[...])


def _tc_forward(gf, gl, W_inf, W_inl, Wq, Wk, Wv, Wo, W1, W2, W_dense,
                b_dense2d, interpret=False):
    B = gf.shape[1]
    BB = 256
    grid = B // BB
    full = lambda shp: pl.BlockSpec(shp, lambda i: tuple(0 for _ in shp))
    return pl.pallas_call(
        _tc_body,
        grid=(grid,),
        in_specs=[
            pl.BlockSpec((T, BB, D), lambda i: (0, i, 0)),
            pl.BlockSpec((T, BB, D), lambda i: (0, i, 0)),
            full((D, EMB)),
            full((D, EMB)),
            full((NLAYER, EMB, EMB)),
            full((NLAYER, EMB, EMB)),
            full((NLAYER, EMB, EMB)),
            full((NLAYER, EMB, EMB)),
            full((NLAYER, FF, EMB)),
            full((NLAYER, EMB, FF)),
            full((EMB, C)),
            full((1, C)),
        ],
        out_specs=pl.BlockSpec((BB, C), lambda i: (i, 0)),
        out_shape=jax.ShapeDtypeStruct((B, C), jnp.float32),
        interpret=interpret,
    )(gf, gl, W_inf, W_inl, Wq, Wk, Wv, Wo, W1, W2, W_dense, b_dense2d)


def kernel(nodes, neigh, feat, lap, W_in, Wq, Wk, Wv, Wo, W1, W2,
           W_dense, b_dense):
    nodes = nodes.astype(jnp.int32)
    neigh = neigh.astype(jnp.int32)
    B = nodes.shape[0]
    neigh_wide = neigh.reshape(neigh.shape[0] * S // 128, 128)
    lap_pad = jnp.pad(lap, ((0, 0), (0, D - DL)))
    W_inf = W_in[:D]
    W_inl = jnp.pad(W_in[D:], ((0, D - DL), (0, 0)))
    nbwide = _sc_gather_nb(nodes, neigh_wide)
    tokT = _tc_tokens(nodes.reshape(1, B), nbwide)
    gf, gl = _sc_gather_rows(tokT, feat, lap_pad)
    return _tc_forward(gf, gl, W_inf, W_inl,
                       jnp.swapaxes(Wq, 1, 2), jnp.swapaxes(Wk, 1, 2),
                       jnp.swapaxes(Wv, 1, 2), jnp.swapaxes(Wo, 1, 2),
                       jnp.swapaxes(W1, 1, 2), jnp.swapaxes(W2, 1, 2),
                       W_dense, b_dense.reshape(1, C))


# speedup vs baseline: 5.9665x; 1.0728x over previous
import functools

import jax
import jax.numpy as jnp
import numpy as np
from jax import lax
from jax.experimental import pallas as pl
from jax.experimental.pallas import tpu as pltpu
from jax.experimental.pallas import tpu_sc as plsc

D = 128
DL = 16
S = 16
T = S + 1
EMB = 128
H = 4
DH = EMB // H
NLAYER = 2
FF = 256
C = 40


def _sc_mesh():
    return plsc.VectorSubcoreMesh(core_axis_name="c", subcore_axis_name="s")


def _worker_info(B):
    info = plsc.get_sparse_core_info()
    NW = info.num_cores * info.num_subcores
    return info.num_cores, NW, B // NW


def _sc_gather_nb(nodes, neigh_wide):
    B = nodes.shape[0]
    NC, NW, BW = _worker_info(B)
    NCHUNK = BW // 128

    @functools.partial(
        pl.kernel,
        mesh=_sc_mesh(),
        out_type=jax.ShapeDtypeStruct((B, 128), jnp.int32),
        scratch_types=[
            pltpu.VMEM((BW,), jnp.int32),
            pltpu.VMEM((BW,), jnp.int32),
            pltpu.VMEM((128, 128), jnp.int32),
            pltpu.SemaphoreType.DMA,
        ],
    )
    def nb_kernel(nodes_hbm, neigh_hbm, nb_hbm, seeds_v, idx_v, buf, sem):
        wid = lax.axis_index("s") * NC + lax.axis_index("c")
        base = wid * BW
        pltpu.sync_copy(nodes_hbm.at[pl.ds(base, BW)], seeds_v)
        for i in range(BW // 16):
            sl = pl.ds(i * 16, 16)
            idx_v[sl] = lax.shift_right_logical(seeds_v[sl], 3)
        for c in range(NCHUNK):
            pltpu.async_copy(
                neigh_hbm.at[idx_v.at[pl.ds(c * 128, 128)]], buf, sem).wait()
            pltpu.sync_copy(buf, nb_hbm.at[pl.ds(base + c * 128, 128), :])

    return nb_kernel(nodes, neigh_wide)


def _tok_body(nodes_ref, nbw_ref, out_ref):
    nodes = nodes_ref[...]
    offs = jnp.transpose(nodes, (1, 0)) & 7
    nb = nbw_ref[:, 0:S]
    for k in range(1, 8):
        nb = jnp.where(offs == k, nbw_ref[:, k * S:(k + 1) * S], nb)
    nT = jnp.transpose(nb, (1, 0))
    out_ref[:, 0, :] = jnp.concatenate([nodes, nT], axis=0)


def _tc_tokens(nodes2d, nbwide):
    B = nbwide.shape[0]
    BB = 256
    return pl.pallas_call(
        _tok_body,
        grid=(B // BB,),
        in_specs=[
            pl.BlockSpec((1, BB), lambda i: (0, i)),
            pl.BlockSpec((BB, 128), lambda i: (i, 0)),
        ],
        out_specs=pl.BlockSpec((T, 1, BB), lambda i: (0, 0, i)),
        out_shape=jax.ShapeDtypeStruct((T, 1, B), jnp.int32),
    )(nodes2d, nbwide)


def _sc_gather_rows(tokT, feat, lap_pad):
    B = tokT.shape[2]
    NC, NW, BW = _worker_info(B)
    NCHUNK = BW // 128

    @functools.partial(
        pl.kernel,
        mesh=_sc_mesh(),
        out_type=[
            jax.ShapeDtypeStruct((T, B, D), jnp.float32),
            jax.ShapeDtypeStruct((T, B, D), jnp.float32),
        ],
        scratch_types=[
            pltpu.VMEM((T * BW,), jnp.int32),
            pltpu.VMEM((2, 128, D), jnp.float32),
            pltpu.VMEM((2, 128, D), jnp.float32),
            pltpu.SemaphoreType.DMA,
            pltpu.SemaphoreType.DMA,
            pltpu.SemaphoreType.DMA,
            pltpu.SemaphoreType.DMA,
        ],
    )
    def row_kernel(tokT_hbm, feat_hbm, lap_hbm, gf_hbm, gl_hbm,
                   tok_v, fbuf, lbuf, fsem, lsem, fwsem, lwsem):
        wid = lax.axis_index("s") * NC + lax.axis_index("c")
        base = wid * BW
        for t in range(T):
            pltpu.sync_copy(tokT_hbm.at[t, 0, pl.ds(base, BW)],
                            tok_v.at[pl.ds(t * BW, BW)])
        steps = [(t, c) for t in range(T) for c in range(NCHUNK)]
        n = len(steps)
        fg = [None] * n
        lg = [None] * n
        fw = [None] * n
        lw = [None] * n

        def start_wb(s):
            t, c = steps[s]
            b = s & 1
            dst = pl.ds(base + c * 128, 128)
            fg[s].wait()
            fw[s] = pltpu.async_copy(fbuf.at[b], gf_hbm.at[t, dst, :], fwsem)
            lg[s].wait()
            lw[s] = pltpu.async_copy(lbuf.at[b], gl_hbm.at[t, dst, :], lwsem)

        for s, (t, c) in enumerate(steps):
            b = s & 1
            if s >= 2:
                fw[s - 2].wait()
                lw[s - 2].wait()
            idx = tok_v.at[pl.ds(t * BW + c * 128, 128)]
            fg[s] = pltpu.async_copy(feat_hbm.at[idx], fbuf.at[b], fsem)
            lg[s] = pltpu.async_copy(lap_hbm.at[idx], lbuf.at[b], lsem)
            if s >= 1:
                start_wb(s - 1)
        start_wb(n - 1)
        fw[n - 2].wait()
        lw[n - 2].wait()
        fw[n - 1].wait()
        lw[n - 1].wait()

    return row_kernel(tokT, feat, lap_pad)


def _layernorm_t(z):
    m = jnp.mean(z, axis=0, keepdims=True)
    zc = z - m
    v = jnp.mean(zc * zc, axis=0, keepdims=True)
    return zc * lax.rsqrt(v + 1e-5)


def _tc_body(gf_ref, gl_ref, winf_ref, winl_ref, wq_ref, wk_ref, wv_ref,
             wo_ref, w1_ref, w2_ref, wd_ref, bd_ref, out_ref):
    BB = gf_ref.shape[1]
    R = T * BB
    scale = np.float32(1.0 / np.sqrt(DH))
    f32 = jnp.float32

    gf = gf_ref[...].reshape(R, D)
    gl = gl_ref[...].reshape(R, D)
    x = (jnp.dot(gf, winf_ref[...], preferred_element_type=f32)
         + jnp.dot(gl, winl_ref[...], preferred_element_type=f32))
    xT = jnp.transpose(x, (1, 0))

    for l in range(NLAYER):
        qT = jnp.dot(wq_ref[l], xT, preferred_element_type=f32)
        kT = jnp.dot(wk_ref[l], xT, preferred_element_type=f32)
        vT = jnp.dot(wv_ref[l], xT, preferred_element_type=f32)
        head_cols = []
        for h in range(H):
            qh = qT[h * DH:(h + 1) * DH, :]
            kh = kT[h * DH:(h + 1) * DH, :]
            vh = vT[h * DH:(h + 1) * DH, :]
            vcols = [vh[:, j * BB:(j + 1) * BB] for j in range(T)]
            icols = []
            for i in range(T):
                qhi = qh[:, i * BB:(i + 1) * BB]
                rows = [jnp.sum(qhi * kh[:, j * BB:(j + 1) * BB],
                                axis=0, keepdims=True) for j in range(T)]
                s = jnp.concatenate(rows, axis=0) * scale
                m = jnp.max(s, axis=0, keepdims=True)
                e = jnp.exp(s - m)
                w = e / jnp.sum(e, axis=0, keepdims=True)
                acc = w[0:1, :] * vcols[0]
                for j in range(1, T):
                    acc = acc + w[j:j + 1, :] * vcols[j]
                icols.append(acc)
            head_cols.append(icols)
        oT = jnp.concatenate(
            [jnp.concatenate(head_cols[h], axis=1) for h in range(H)], axis=0)
        xT = _layernorm_t(xT + jnp.dot(wo_ref[l], oT,
                                       preferred_element_type=f32))
        f = jnp.dot(w2_ref[l],
                    jax.nn.relu(jnp.dot(w1_ref[l], xT,
                                        preferred_element_type=f32)),
                    preferred_element_type=f32)
        xT = _layernorm_t(xT + f)

    h0 = jnp.transpose(xT[:, :BB], (1, 0))
    out_ref[...] = (jnp.dot(h0, wd_ref[...], preferred_element_type=f32)
                    + bd_ref[...])


def _tc_forward(gf, gl, W_inf, W_inl, Wq, Wk, Wv, Wo, W1, W2, W_dense,
                b_dense2d, interpret=False):
    B = gf.shape[1]
    BB = 256
    grid = B // BB
    full = lambda shp: pl.BlockSpec(shp, lambda i: tuple(0 for _ in shp))
    return pl.pallas_call(
        _tc_body,
        grid=(grid,),
        in_specs=[
            pl.BlockSpec((T, BB, D), lambda i: (0, i, 0)),
            pl.BlockSpec((T, BB, D), lambda i: (0, i, 0)),
            full((D, EMB)),
            full((D, EMB)),
            full((NLAYER, EMB, EMB)),
            full((NLAYER, EMB, EMB)),
            full((NLAYER, EMB, EMB)),
            full((NLAYER, EMB, EMB)),
            full((NLAYER, FF, EMB)),
            full((NLAYER, EMB, FF)),
            full((EMB, C)),
            full((1, C)),
        ],
        out_specs=pl.BlockSpec((BB, C), lambda i: (i, 0)),
        out_shape=jax.ShapeDtypeStruct((B, C), jnp.float32),
        interpret=interpret,
    )(gf, gl, W_inf, W_inl, Wq, Wk, Wv, Wo, W1, W2, W_dense, b_dense2d)


def kernel(nodes, neigh, feat, lap, W_in, Wq, Wk, Wv, Wo, W1, W2,
           W_dense, b_dense):
    nodes = nodes.astype(jnp.int32)
    neigh = neigh.astype(jnp.int32)
    B = nodes.shape[0]
    neigh_wide = neigh.reshape(neigh.shape[0] * S // 128, 128)
    lap_pad = jnp.pad(lap, ((0, 0), (0, D - DL)))
    W_inf = W_in[:D]
    W_inl = jnp.pad(W_in[D:], ((0, D - DL), (0, 0)))
    nbwide = _sc_gather_nb(nodes, neigh_wide)
    tokT = _tc_tokens(nodes.reshape(1, B), nbwide)
    wts = (W_inf, W_inl,
           jnp.swapaxes(Wq, 1, 2), jnp.swapaxes(Wk, 1, 2),
           jnp.swapaxes(Wv, 1, 2), jnp.swapaxes(Wo, 1, 2),
           jnp.swapaxes(W1, 1, 2), jnp.swapaxes(W2, 1, 2),
           W_dense, b_dense.reshape(1, C))
    NSPLIT = 2
    BH = B // NSPLIT
    outs = []
    for hch in range(NSPLIT):
        tokT_h = tokT[:, :, hch * BH:(hch + 1) * BH]
        gf, gl = _sc_gather_rows(tokT_h, feat, lap_pad)
        outs.append(_tc_forward(gf, gl, *wts))
    return jnp.concatenate(outs, axis=0)


# speedup vs baseline: 5.9857x; 1.0032x over previous
import functools

import jax
import jax.numpy as jnp
import numpy as np
from jax import lax
from jax.experimental import pallas as pl
from jax.experimental.pallas import tpu as pltpu
from jax.experimental.pallas import tpu_sc as plsc

D = 128
DL = 16
S = 16
T = S + 1
EMB = 128
H = 4
DH = EMB // H
NLAYER = 2
FF = 256
C = 40


def _sc_mesh():
    return plsc.VectorSubcoreMesh(core_axis_name="c", subcore_axis_name="s")


def _worker_info(B):
    info = plsc.get_sparse_core_info()
    NW = info.num_cores * info.num_subcores
    return info.num_cores, NW, B // NW


def _sc_gather_nb(nodes, neigh_wide):
    B = nodes.shape[0]
    NC, NW, BW = _worker_info(B)
    NCHUNK = BW // 128

    @functools.partial(
        pl.kernel,
        mesh=_sc_mesh(),
        out_type=jax.ShapeDtypeStruct((B, 128), jnp.int32),
        scratch_types=[
            pltpu.VMEM((BW,), jnp.int32),
            pltpu.VMEM((BW,), jnp.int32),
            pltpu.VMEM((128, 128), jnp.int32),
            pltpu.SemaphoreType.DMA,
        ],
    )
    def nb_kernel(nodes_hbm, neigh_hbm, nb_hbm, seeds_v, idx_v, buf, sem):
        wid = lax.axis_index("s") * NC + lax.axis_index("c")
        base = wid * BW
        pltpu.sync_copy(nodes_hbm.at[pl.ds(base, BW)], seeds_v)
        for i in range(BW // 16):
            sl = pl.ds(i * 16, 16)
            idx_v[sl] = lax.shift_right_logical(seeds_v[sl], 3)
        for c in range(NCHUNK):
            pltpu.async_copy(
                neigh_hbm.at[idx_v.at[pl.ds(c * 128, 128)]], buf, sem).wait()
            pltpu.sync_copy(buf, nb_hbm.at[pl.ds(base + c * 128, 128), :])

    return nb_kernel(nodes, neigh_wide)


def _tok_body(nodes_ref, nbw_ref, out_ref):
    nodes = nodes_ref[...]
    offs = jnp.transpose(nodes, (1, 0)) & 7
    nb = nbw_ref[:, 0:S]
    for k in range(1, 8):
        nb = jnp.where(offs == k, nbw_ref[:, k * S:(k + 1) * S], nb)
    nT = jnp.transpose(nb, (1, 0))
    out_ref[:, 0, :] = jnp.concatenate([nodes, nT], axis=0)


def _tc_tokens(nodes2d, nbwide):
    B = nbwide.shape[0]
    BB = 256
    return pl.pallas_call(
        _tok_body,
        grid=(B // BB,),
        in_specs=[
            pl.BlockSpec((1, BB), lambda i: (0, i)),
            pl.BlockSpec((BB, 128), lambda i: (i, 0)),
        ],
        out_specs=pl.BlockSpec((T, 1, BB), lambda i: (0, 0, i)),
        out_shape=jax.ShapeDtypeStruct((T, 1, B), jnp.int32),
    )(nodes2d, nbwide)


def _sc_gather_rows(tokT, feat, lap_pad):
    B = tokT.shape[2]
    NC, NW, BW = _worker_info(B)
    CH = min(BW, 128)
    NCHUNK = BW // CH

    @functools.partial(
        pl.kernel,
        mesh=_sc_mesh(),
        out_type=[
            jax.ShapeDtypeStruct((T, B, D), jnp.float32),
            jax.ShapeDtypeStruct((T, B, D), jnp.float32),
        ],
        scratch_types=[
            pltpu.VMEM((T * BW,), jnp.int32),
            pltpu.VMEM((2, CH, D), jnp.float32),
            pltpu.VMEM((2, CH, D), jnp.float32),
            pltpu.SemaphoreType.DMA,
            pltpu.SemaphoreType.DMA,
            pltpu.SemaphoreType.DMA,
            pltpu.SemaphoreType.DMA,
        ],
    )
    def row_kernel(tokT_hbm, feat_hbm, lap_hbm, gf_hbm, gl_hbm,
                   tok_v, fbuf, lbuf, fsem, lsem, fwsem, lwsem):
        wid = lax.axis_index("s") * NC + lax.axis_index("c")
        base = wid * BW
        for t in range(T):
            pltpu.sync_copy(tokT_hbm.at[t, 0, pl.ds(base, BW)],
                            tok_v.at[pl.ds(t * BW, BW)])
        steps = [(t, c) for t in range(T) for c in range(NCHUNK)]
        n = len(steps)
        fg = [None] * n
        lg = [None] * n
        fw = [None] * n
        lw = [None] * n

        def start_wb(s):
            t, c = steps[s]
            b = s & 1
            dst = pl.ds(base + c * CH, CH)
            fg[s].wait()
            fw[s] = pltpu.async_copy(fbuf.at[b], gf_hbm.at[t, dst, :], fwsem)
            lg[s].wait()
            lw[s] = pltpu.async_copy(lbuf.at[b], gl_hbm.at[t, dst, :], lwsem)

        for s, (t, c) in enumerate(steps):
            b = s & 1
            if s >= 2:
                fw[s - 2].wait()
                lw[s - 2].wait()
            idx = tok_v.at[pl.ds(t * BW + c * CH, CH)]
            fg[s] = pltpu.async_copy(feat_hbm.at[idx], fbuf.at[b], fsem)
            lg[s] = pltpu.async_copy(lap_hbm.at[idx], lbuf.at[b], lsem)
            if s >= 1:
                start_wb(s - 1)
        start_wb(n - 1)
        fw[n - 2].wait()
        lw[n - 2].wait()
        fw[n - 1].wait()
        lw[n - 1].wait()

    return row_kernel(tokT, feat, lap_pad)


def _layernorm_t(z):
    m = jnp.mean(z, axis=0, keepdims=True)
    zc = z - m
    v = jnp.mean(zc * zc, axis=0, keepdims=True)
    return zc * lax.rsqrt(v + 1e-5)


def _tc_body(gf_ref, gl_ref, winf_ref, winl_ref, wq_ref, wk_ref, wv_ref,
             wo_ref, w1_ref, w2_ref, wd_ref, bd_ref, out_ref):
    BB = gf_ref.shape[1]
    R = T * BB
    scale = np.float32(1.0 / np.sqrt(DH))
    f32 = jnp.float32

    gf = gf_ref[...].reshape(R, D)
    gl = gl_ref[...].reshape(R, D)
    x = (jnp.dot(gf, winf_ref[...], preferred_element_type=f32)
         + jnp.dot(gl, winl_ref[...], preferred_element_type=f32))
    xT = jnp.transpose(x, (1, 0))

    for l in range(NLAYER):
        qT = jnp.dot(wq_ref[l], xT, preferred_element_type=f32)
        kT = jnp.dot(wk_ref[l], xT, preferred_element_type=f32)
        vT = jnp.dot(wv_ref[l], xT, preferred_element_type=f32)
        head_cols = []
        for h in range(H):
            qh = qT[h * DH:(h + 1) * DH, :]
            kh = kT[h * DH:(h + 1) * DH, :]
            vh = vT[h * DH:(h + 1) * DH, :]
            vcols = [vh[:, j * BB:(j + 1) * BB] for j in range(T)]
            icols = []
            for i in range(T):
                qhi = qh[:, i * BB:(i + 1) * BB]
                rows = [jnp.sum(qhi * kh[:, j * BB:(j + 1) * BB],
                                axis=0, keepdims=True) for j in range(T)]
                s = jnp.concatenate(rows, axis=0) * scale
                m = jnp.max(s, axis=0, keepdims=True)
                e = jnp.exp(s - m)
                w = e / jnp.sum(e, axis=0, keepdims=True)
                acc = w[0:1, :] * vcols[0]
                for j in range(1, T):
                    acc = acc + w[j:j + 1, :] * vcols[j]
                icols.append(acc)
            head_cols.append(icols)
        oT = jnp.concatenate(
            [jnp.concatenate(head_cols[h], axis=1) for h in range(H)], axis=0)
        xT = _layernorm_t(xT + jnp.dot(wo_ref[l], oT,
                                       preferred_element_type=f32))
        f = jnp.dot(w2_ref[l],
                    jax.nn.relu(jnp.dot(w1_ref[l], xT,
                                        preferred_element_type=f32)),
                    preferred_element_type=f32)
        xT = _layernorm_t(xT + f)

    h0 = jnp.transpose(xT[:, :BB], (1, 0))
    out_ref[...] = (jnp.dot(h0, wd_ref[...], preferred_element_type=f32)
                    + bd_ref[...])


def _tc_forward(gf, gl, W_inf, W_inl, Wq, Wk, Wv, Wo, W1, W2, W_dense,
                b_dense2d, interpret=False):
    B = gf.shape[1]
    BB = 256
    grid = B // BB
    full = lambda shp: pl.BlockSpec(shp, lambda i: tuple(0 for _ in shp))
    return pl.pallas_call(
        _tc_body,
        grid=(grid,),
        in_specs=[
            pl.BlockSpec((T, BB, D), lambda i: (0, i, 0)),
            pl.BlockSpec((T, BB, D), lambda i: (0, i, 0)),
            full((D, EMB)),
            full((D, EMB)),
            full((NLAYER, EMB, EMB)),
            full((NLAYER, EMB, EMB)),
            full((NLAYER, EMB, EMB)),
            full((NLAYER, EMB, EMB)),
            full((NLAYER, FF, EMB)),
            full((NLAYER, EMB, FF)),
            full((EMB, C)),
            full((1, C)),
        ],
        out_specs=pl.BlockSpec((BB, C), lambda i: (i, 0)),
        out_shape=jax.ShapeDtypeStruct((B, C), jnp.float32),
        interpret=interpret,
    )(gf, gl, W_inf, W_inl, Wq, Wk, Wv, Wo, W1, W2, W_dense, b_dense2d)


def kernel(nodes, neigh, feat, lap, W_in, Wq, Wk, Wv, Wo, W1, W2,
           W_dense, b_dense):
    nodes = nodes.astype(jnp.int32)
    neigh = neigh.astype(jnp.int32)
    B = nodes.shape[0]
    neigh_wide = neigh.reshape(neigh.shape[0] * S // 128, 128)
    lap_pad = jnp.pad(lap, ((0, 0), (0, D - DL)))
    W_inf = W_in[:D]
    W_inl = jnp.pad(W_in[D:], ((0, D - DL), (0, 0)))
    nbwide = _sc_gather_nb(nodes, neigh_wide)
    tokT = _tc_tokens(nodes.reshape(1, B), nbwide)
    wts = (W_inf, W_inl,
           jnp.swapaxes(Wq, 1, 2), jnp.swapaxes(Wk, 1, 2),
           jnp.swapaxes(Wv, 1, 2), jnp.swapaxes(Wo, 1, 2),
           jnp.swapaxes(W1, 1, 2), jnp.swapaxes(W2, 1, 2),
           W_dense, b_dense.reshape(1, C))
    NSPLIT = 4
    BH = B // NSPLIT
    outs = []
    for hch in range(NSPLIT):
        tokT_h = tokT[:, :, hch * BH:(hch + 1) * BH]
        gf, gl = _sc_gather_rows(tokT_h, feat, lap_pad)
        outs.append(_tc_forward(gf, gl, *wts))
    return jnp.concatenate(outs, axis=0)
